# baseline (speedup 1.0000x reference)
import math
import functools

import jax
import jax.numpy as jnp
from jax import lax
from jax.experimental import pallas as pl
from jax.experimental.pallas import tpu as pltpu
from jax.experimental.pallas import tpu_sc as plsc

NEG = -1e30
LN10K = math.log(10000.0)


def _rope_tables(nrows, half):
    pos = jnp.arange(nrows, dtype=jnp.float32)[:, None]
    j = jnp.arange(half, dtype=jnp.float32)[None, :]
    ang = pos * jnp.exp(j * (-LN10K / half))
    cos, sin = jnp.cos(ang), jnp.sin(ang)
    return (jnp.concatenate([cos, cos], axis=-1),
            jnp.concatenate([-sin, sin], axis=-1))


def _rms(x, w, eps=1e-6):
    return x * lax.rsqrt(jnp.mean(x * x, axis=-1, keepdims=True) + eps) * w


def _halfswap(x):
    half = x.shape[-1] // 2
    return jnp.concatenate([x[:, half:], x[:, :half]], axis=-1)


def _rope_fwd(x, cos2, snpm):
    return x * cos2 + _halfswap(x) * snpm


def _rope_inv(x, cos2, snpm):
    return x * cos2 - _halfswap(x) * snpm


def _mmt(a, b):
    return lax.dot_general(a, b, (((1,), (1,)), ((), ())),
                           preferred_element_type=jnp.float32)


def _prep_kernel(A_ref, H_ref, Wc0_ref, Wc1_ref, Wi0_ref, Wi1_ref,
                 Wkv_ref, cb_ref, ib_ref, kw_ref, vw_ref, cos_ref, sin_ref,
                 KI_ref, KV_ref):
    A = A_ref[...]
    KI_ref[...] = (jnp.dot(A[:512], Wi0_ref[...], preferred_element_type=jnp.float32)
                   + jnp.dot(A[1:], Wi1_ref[...], preferred_element_type=jnp.float32)
                   + ib_ref[...])
    kv_comp = (jnp.dot(A[:512], Wc0_ref[...], preferred_element_type=jnp.float32)
               + jnp.dot(A[1:], Wc1_ref[...], preferred_element_type=jnp.float32)
               + cb_ref[...])
    kv_slide = jnp.dot(H_ref[...], Wkv_ref[...], preferred_element_type=jnp.float32)
    kv = jnp.concatenate([kv_comp, kv_slide], axis=0)
    rs = lax.rsqrt(jnp.mean(kv * kv, axis=-1, keepdims=True) + 1e-6)
    cos2 = cos_ref[...]
    snpm = sin_ref[...]
    K = _rope_fwd(kv * rs * kw_ref[...], cos2, snpm)
    V = _rope_fwd(kv * rs * vw_ref[...], cos2, snpm)
    KV_ref[...] = jnp.concatenate([K, V], axis=1)


def _select_kernel(H_ref, KI_ref, Wdq_ref, Wiuq_ref, Ww_ref, idx_ref,
                   *, Tq, Tc):
    i = pl.program_id(0)
    t0 = i * Tq
    Hb = H_ref[...]
    h_dc = jnp.dot(Hb, Wdq_ref[...], preferred_element_type=jnp.float32)
    QI = jnp.dot(h_dc, Wiuq_ref[...], preferred_element_type=jnp.float32)
    WI = jnp.dot(h_dc, Ww_ref[...], preferred_element_type=jnp.float32)
    KI = KI_ref[...]

    I_ts = jnp.zeros((Tq, Tc), jnp.float32)
    for h in range(4):
        d_h = _mmt(QI[:, 32 * h:32 * h + 32], KI)
        I_ts = I_ts + WI[:, h:h + 1] * jnp.maximum(d_h, 0.0)

    row = lax.broadcasted_iota(jnp.int32, (Tq, Tc), 0)
    col = lax.broadcasted_iota(jnp.int32, (Tq, Tc), 1)
    colf = col.astype(jnp.float32)
    valid = (4 * col) <= (row + t0)
    scores = jnp.where(valid, I_ts, NEG)

    for j in range(8):
        m = jnp.max(scores, axis=1, keepdims=True)
        idx = jnp.min(jnp.where(scores == m, colf, float(Tc)),
                      axis=1, keepdims=True)
        scores = jnp.where(colf == idx, -3e30, scores)
        idx_ref[:, j:j + 1] = idx.astype(jnp.int32)


def _attn_kernel(H_ref, KV_ref, KVg_ref, cos_ref, sin_ref,
                 Wq_ref, qw_ref,
                 g0W_ref, g0b_ref, g1W_ref, g1b_ref, oW_ref, ob_ref,
                 out_ref, *, Tq, Tc, n_win):
    i = pl.program_id(0)
    t0 = i * Tq
    Hb = H_ref[...]

    cos_q = cos_ref[...]
    sin_q = sin_ref[...]
    c = KV_ref.shape[1] // 2
    s0 = Tc + t0 - n_win
    KVs = KV_ref[pl.ds(s0, Tq + n_win), :]
    Ks = KVs[:, :c]
    Vs = KVs[:, c:]

    rs = lax.broadcasted_iota(jnp.int32, (Tq, Tq + n_win), 0)
    cs = lax.broadcasted_iota(jnp.int32, (Tq, Tq + n_win), 1)
    band = (cs >= rs + 1) & (cs <= rs + n_win) & (cs + t0 >= n_win)

    scale = 1.0 / math.sqrt(float(c))
    bd_r = lax.broadcasted_iota(jnp.int32, (8 * c, 8), 0)
    bd_c = lax.broadcasted_iota(jnp.int32, (8 * c, 8), 1)
    BD = ((bd_r // c) == bd_c).astype(jnp.float32)
    Kg = [KVg_ref[j][:, :c] for j in range(8)]
    Vg = [KVg_ref[j][:, c:] for j in range(8)]
    Qall = jnp.dot(Hb, Wq_ref[...], preferred_element_type=jnp.float32)
    O_heads = []
    for h in range(4):
        qh = _rms(Qall[:, 64 * h:64 * h + 64], qw_ref[h:h + 1, :])
        qh = _rope_fwd(qh, cos_q, sin_q)
        prods = jnp.concatenate([qh * Kg[j] for j in range(8)], axis=1)
        s_comp = jnp.dot(prods, BD,
                         preferred_element_type=jnp.float32) * scale
        s_sl = jnp.where(band, _mmt(qh, Ks) * scale, NEG)
        mx = jnp.maximum(jnp.max(s_comp, axis=1, keepdims=True),
                         jnp.max(s_sl, axis=1, keepdims=True))
        pc = jnp.exp(s_comp - mx)
        ps = jnp.exp(s_sl - mx)
        den = (jnp.sum(pc, axis=1, keepdims=True)
               + jnp.sum(ps, axis=1, keepdims=True))
        o = jnp.dot(ps, Vs, preferred_element_type=jnp.float32)
        for j in range(8):
            o = o + pc[:, j:j + 1] * Vg[j]
        o = o / den
        O_heads.append(_rope_inv(o, cos_q, sin_q))

    og0 = jnp.concatenate([O_heads[0], O_heads[1]], axis=1)
    og1 = jnp.concatenate([O_heads[2], O_heads[3]], axis=1)
    p0 = jnp.dot(og0, g0W_ref[...], preferred_element_type=jnp.float32) + g0b_ref[...]
    p1 = jnp.dot(og1, g1W_ref[...], preferred_element_type=jnp.float32) + g1b_ref[...]
    p = jnp.concatenate([p0, p1], axis=1)
    out_ref[...] = jnp.dot(p, oW_ref[...], preferred_element_type=jnp.float32) + ob_ref[...]


def _sc_gather(KV, idx_flat, n_rows, row_w):
    info = plsc.get_sparse_core_info()
    nw = info.num_cores * info.num_subcores
    per_w = n_rows // nw
    n_chunks = per_w // 128
    mesh = plsc.VectorSubcoreMesh(core_axis_name="c", subcore_axis_name="s")

    @functools.partial(
        pl.kernel, mesh=mesh,
        out_type=jax.ShapeDtypeStruct((n_rows, row_w), jnp.float32),
        scratch_types=[pltpu.VMEM((per_w,), jnp.int32),
                       pltpu.VMEM((per_w, row_w), jnp.float32),
                       pltpu.SemaphoreType.DMA],
    )
    def gather(KV_hbm, idx_hbm, out_hbm, idx_v, rows_v, sem):
        wid = lax.axis_index("s") * info.num_cores + lax.axis_index("c")
        base = wid * per_w
        pltpu.sync_copy(idx_hbm.at[pl.ds(base, per_w)], idx_v)
        copies = [
            pltpu.async_copy(KV_hbm.at[idx_v.at[pl.ds(128 * j, 128)]],
                             rows_v.at[pl.ds(128 * j, 128), :], sem)
            for j in range(n_chunks)
        ]
        for cp in copies:
            cp.wait()
        pltpu.sync_copy(rows_v, out_hbm.at[pl.ds(base, per_w)])

    return gather(KV, idx_flat)


def kernel(H, comp_W, comp_b, idx_W, idx_b, W_DQ, W_IUQ, W_w, W_Q, W_KV,
           rms_q_w, rms_k_w, rms_v_w, g0_W, g0_b, g1_W, g1_b, out_W, out_b):
    B, T, d = H.shape
    c = rms_k_w.shape[0]
    Tc = T // 4
    n_win = 16
    Tq = 256

    H2 = H[0]
    A = jnp.pad(H2, ((2, 2), (0, 0))).reshape(T // 4 + 1, 4 * d)
    Wc = comp_W.reshape(8 * d, c)
    Wi = idx_W.reshape(8 * d, idx_W.shape[2])

    COS, SIN = _rope_tables(Tc + T, c // 2)
    KI, KV = pl.pallas_call(
        _prep_kernel,
        out_shape=[
            jax.ShapeDtypeStruct((Tc, Wi.shape[1]), jnp.float32),
            jax.ShapeDtypeStruct((Tc + T, 2 * c), jnp.float32),
        ],
    )(A, H2, Wc[:4 * d], Wc[4 * d:], Wi[:4 * d], Wi[4 * d:],
      W_KV, comp_b.reshape(1, c), idx_b.reshape(1, -1),
      rms_k_w.reshape(1, c), rms_v_w.reshape(1, c), COS, SIN)

    nblk = T // Tq
    full = lambda arr: pl.BlockSpec(arr.shape, lambda i: (0,) * arr.ndim)

    idx_qm = pl.pallas_call(
        functools.partial(_select_kernel, Tq=Tq, Tc=Tc),
        grid=(nblk,),
        in_specs=[
            pl.BlockSpec((Tq, d), lambda i: (i, 0)),
            full(KI),
            full(W_DQ), full(W_IUQ), full(W_w),
        ],
        out_specs=pl.BlockSpec((Tq, 8), lambda i: (i, 0)),
        out_shape=jax.ShapeDtypeStruct((T, 8), jnp.int32),
    )(H2, KI, W_DQ, W_IUQ, W_w)

    idx_flat = idx_qm.T.reshape(8 * T)
    KVg = _sc_gather(KV, idx_flat, 8 * T, 2 * c).reshape(8, T, 2 * c)

    out = pl.pallas_call(
        functools.partial(_attn_kernel, Tq=Tq, Tc=Tc, n_win=n_win),
        grid=(nblk,),
        in_specs=[
            pl.BlockSpec((Tq, d), lambda i: (i, 0)),
            full(KV),
            pl.BlockSpec((8, Tq, 2 * c), lambda i: (0, i, 0)),
            pl.BlockSpec((Tq, c), lambda i: (i, 0)),
            pl.BlockSpec((Tq, c), lambda i: (i, 0)),
            full(W_Q), full(rms_q_w),
            full(g0_W), pl.BlockSpec((1, g0_W.shape[1]), lambda i: (0, 0)),
            full(g1_W), pl.BlockSpec((1, g1_W.shape[1]), lambda i: (0, 0)),
            full(out_W), pl.BlockSpec((1, d), lambda i: (0, 0)),
        ],
        out_specs=pl.BlockSpec((Tq, d), lambda i: (i, 0)),
        out_shape=jax.ShapeDtypeStruct((T, d), jnp.float32),
    )(H2, KV, KVg, COS, SIN, W_Q, rms_q_w,
      g0_W, g0_b.reshape(1, -1), g1_W, g1_b.reshape(1, -1),
      out_W, out_b.reshape(1, -1))

    return out[None]


# speedup vs baseline: 1.7802x; 1.7802x over previous
import math
import functools

import jax
import jax.numpy as jnp
from jax import lax
from jax.experimental import pallas as pl

NEG = -1e30
LN10K = math.log(10000.0)


def _rope_tables(nrows, half):
    pos = jnp.arange(nrows, dtype=jnp.float32)[:, None]
    j = jnp.arange(half, dtype=jnp.float32)[None, :]
    ang = pos * jnp.exp(j * (-LN10K / half))
    cos, sin = jnp.cos(ang), jnp.sin(ang)
    return (jnp.concatenate([cos, cos], axis=-1),
            jnp.concatenate([-sin, sin], axis=-1))


def _rms(x, w, eps=1e-6):
    return x * lax.rsqrt(jnp.mean(x * x, axis=-1, keepdims=True) + eps) * w


def _halfswap(x):
    half = x.shape[-1] // 2
    return jnp.concatenate([x[:, half:], x[:, :half]], axis=-1)


def _rope_fwd(x, cos2, snpm):
    return x * cos2 + _halfswap(x) * snpm


def _rope_inv(x, cos2, snpm):
    return x * cos2 - _halfswap(x) * snpm


def _mmt(a, b):
    return lax.dot_general(a, b, (((1,), (1,)), ((), ())),
                           preferred_element_type=jnp.float32)


def _prep_kernel(A_ref, H_ref, Wc0_ref, Wc1_ref, Wi0_ref, Wi1_ref,
                 Wkv_ref, cb_ref, ib_ref, kw_ref, vw_ref, cos_ref, sin_ref,
                 KI_ref, K_ref, V_ref):
    A = A_ref[...]
    KI_ref[...] = (jnp.dot(A[:512], Wi0_ref[...], preferred_element_type=jnp.float32)
                   + jnp.dot(A[1:], Wi1_ref[...], preferred_element_type=jnp.float32)
                   + ib_ref[...])
    kv_comp = (jnp.dot(A[:512], Wc0_ref[...], preferred_element_type=jnp.float32)
               + jnp.dot(A[1:], Wc1_ref[...], preferred_element_type=jnp.float32)
               + cb_ref[...])
    kv_slide = jnp.dot(H_ref[...], Wkv_ref[...], preferred_element_type=jnp.float32)
    kv = jnp.concatenate([kv_comp, kv_slide], axis=0)
    rs = lax.rsqrt(jnp.mean(kv * kv, axis=-1, keepdims=True) + 1e-6)
    cos2 = cos_ref[...]
    snpm = sin_ref[...]
    K_ref[...] = _rope_fwd(kv * rs * kw_ref[...], cos2, snpm)
    V_ref[...] = _rope_fwd(kv * rs * vw_ref[...], cos2, snpm)


def _attn_kernel(H_ref, KI_ref, Kc_ref, Vc_ref, Km_ref, Vm_ref,
                 Kh_ref, Vh_ref, cos_ref, sin_ref,
                 Wdq_ref, Wiuq_ref, Ww_ref, Wq_ref, qw_ref,
                 g0W_ref, g0b_ref, g1W_ref, g1b_ref, oW_ref, ob_ref,
                 out_ref, *, Tq, Tc, n_win):
    i = pl.program_id(0)
    t0 = i * Tq
    Hb = H_ref[...]
    h_dc = jnp.dot(Hb, Wdq_ref[...], preferred_element_type=jnp.float32)
    QI = jnp.dot(h_dc, Wiuq_ref[...], preferred_element_type=jnp.float32)
    WI = jnp.dot(h_dc, Ww_ref[...], preferred_element_type=jnp.float32)
    KI = KI_ref[...]

    I_ts = jnp.zeros((Tq, Tc), jnp.float32)
    for h in range(4):
        d_h = _mmt(QI[:, 32 * h:32 * h + 32], KI)
        I_ts = I_ts + WI[:, h:h + 1] * jnp.maximum(d_h, 0.0)

    row = lax.broadcasted_iota(jnp.int32, (Tq, Tc), 0)
    col = lax.broadcasted_iota(jnp.int32, (Tq, Tc), 1)
    colf = col.astype(jnp.float32)
    valid = (4 * col) <= (row + t0)
    scores = jnp.where(valid, I_ts, NEG)

    for _ in range(8):
        m = jnp.max(scores, axis=1, keepdims=True)
        idx = jnp.min(jnp.where(scores == m, colf, float(Tc)),
                      axis=1, keepdims=True)
        scores = jnp.where(colf == idx, -3e30, scores)
    M = jnp.where(scores == -3e30, 0.0, NEG)

    cos_q = cos_ref[...]
    sin_q = sin_ref[...]
    Kc = Kc_ref[...]
    Vc = Vc_ref[...]
    Ks = jnp.concatenate([Kh_ref[...], Km_ref[...]], axis=0)
    Vs = jnp.concatenate([Vh_ref[...], Vm_ref[...]], axis=0)

    rs = lax.broadcasted_iota(jnp.int32, (Tq, Tq + n_win), 0)
    cs = lax.broadcasted_iota(jnp.int32, (Tq, Tq + n_win), 1)
    band = (cs >= rs + 1) & (cs <= rs + n_win) & (cs + t0 >= n_win)

    scale = 1.0 / math.sqrt(float(Kc_ref.shape[1]))
    Qall = jnp.dot(Hb, Wq_ref[...], preferred_element_type=jnp.float32)
    O_heads = []
    for h in range(4):
        qh = _rms(Qall[:, 64 * h:64 * h + 64], qw_ref[h:h + 1, :])
        qh = _rope_fwd(qh, cos_q, sin_q)
        s_comp = _mmt(qh, Kc) * scale + M
        s_sl = jnp.where(band, _mmt(qh, Ks) * scale, NEG)
        mx = jnp.maximum(jnp.max(s_comp, axis=1, keepdims=True),
                         jnp.max(s_sl, axis=1, keepdims=True))
        pc = jnp.exp(s_comp - mx)
        ps = jnp.exp(s_sl - mx)
        den = (jnp.sum(pc, axis=1, keepdims=True)
               + jnp.sum(ps, axis=1, keepdims=True))
        o = (jnp.dot(pc, Vc, preferred_element_type=jnp.float32)
             + jnp.dot(ps, Vs, preferred_element_type=jnp.float32)) / den
        O_heads.append(_rope_inv(o, cos_q, sin_q))

    og0 = jnp.concatenate([O_heads[0], O_heads[1]], axis=1)
    og1 = jnp.concatenate([O_heads[2], O_heads[3]], axis=1)
    p0 = jnp.dot(og0, g0W_ref[...], preferred_element_type=jnp.float32) + g0b_ref[...]
    p1 = jnp.dot(og1, g1W_ref[...], preferred_element_type=jnp.float32) + g1b_ref[...]
    p = jnp.concatenate([p0, p1], axis=1)
    out_ref[...] = jnp.dot(p, oW_ref[...], preferred_element_type=jnp.float32) + ob_ref[...]


def kernel(H, comp_W, comp_b, idx_W, idx_b, W_DQ, W_IUQ, W_w, W_Q, W_KV,
           rms_q_w, rms_k_w, rms_v_w, g0_W, g0_b, g1_W, g1_b, out_W, out_b):
    B, T, d = H.shape
    c = rms_k_w.shape[0]
    Tc = T // 4
    n_win = 16
    Tq = 256

    H2 = H[0]
    A = jnp.pad(H2, ((2, 2), (0, 0))).reshape(T // 4 + 1, 4 * d)
    Wc = comp_W.reshape(8 * d, c)
    Wi = idx_W.reshape(8 * d, idx_W.shape[2])

    COS, SIN = _rope_tables(Tc + T, c // 2)
    KI, K, V = pl.pallas_call(
        _prep_kernel,
        out_shape=[
            jax.ShapeDtypeStruct((Tc, Wi.shape[1]), jnp.float32),
            jax.ShapeDtypeStruct((Tc + T, c), jnp.float32),
            jax.ShapeDtypeStruct((Tc + T, c), jnp.float32),
        ],
    )(A, H2, Wc[:4 * d], Wc[4 * d:], Wi[:4 * d], Wi[4 * d:],
      W_KV, comp_b.reshape(1, c), idx_b.reshape(1, -1),
      rms_k_w.reshape(1, c), rms_v_w.reshape(1, c), COS, SIN)

    nblk = T // Tq
    full = lambda arr: pl.BlockSpec(arr.shape, lambda i: (0,) * arr.ndim)
    out = pl.pallas_call(
        functools.partial(_attn_kernel, Tq=Tq, Tc=Tc, n_win=n_win),
        grid=(nblk,),
        in_specs=[
            pl.BlockSpec((Tq, d), lambda i: (i, 0)),
            full(KI),
            pl.BlockSpec((Tc, c), lambda i: (0, 0)),
            pl.BlockSpec((Tc, c), lambda i: (0, 0)),
            pl.BlockSpec((Tq, c), lambda i: (Tc // Tq + i, 0)),
            pl.BlockSpec((Tq, c), lambda i: (Tc // Tq + i, 0)),
            pl.BlockSpec((n_win, c),
                         lambda i: ((Tc - n_win) // n_win + (Tq // n_win) * i, 0)),
            pl.BlockSpec((n_win, c),
                         lambda i: ((Tc - n_win) // n_win + (Tq // n_win) * i, 0)),
            pl.BlockSpec((Tq, c), lambda i: (i, 0)),
            pl.BlockSpec((Tq, c), lambda i: (i, 0)),
            full(W_DQ), full(W_IUQ), full(W_w), full(W_Q), full(rms_q_w),
            full(g0_W), pl.BlockSpec((1, g0_W.shape[1]), lambda i: (0, 0)),
            full(g1_W), pl.BlockSpec((1, g1_W.shape[1]), lambda i: (0, 0)),
            full(out_W), pl.BlockSpec((1, d), lambda i: (0, 0)),
        ],
        out_specs=pl.BlockSpec((Tq, d), lambda i: (i, 0)),
        out_shape=jax.ShapeDtypeStruct((T, d), jnp.float32),
    )(H2, KI, K, V, K, V, K, V, COS, SIN, W_DQ, W_IUQ, W_w, W_Q, rms_q_w,
      g0_W, g0_b.reshape(1, -1), g1_W, g1_b.reshape(1, -1),
      out_W, out_b.reshape(1, -1))

    return out[None]
